# baseline (speedup 1.0000x reference)
import functools

import jax
import jax.numpy as jnp
from jax import lax
from jax.experimental import pallas as pl
from jax.experimental.pallas import tpu as pltpu
from jax.experimental.pallas import tpu_sc as plsc

NNZ = 2097152
D = 32
NV = 1000000
NC = 2
NS = 16
NW = NC * NS
PER_W = NNZ // NW
C = 256
NCHUNK = PER_W // C

BW = 256
NBLK = 3904
MAIN = NBLK * BW
TAIL = NV - MAIN
NB_W = NBLK // NW

_mesh = plsc.VectorSubcoreMesh(core_axis_name="c", subcore_axis_name="s")


@functools.partial(
    pl.kernel,
    mesh=_mesh,
    compiler_params=pltpu.CompilerParams(needs_layout_passes=False),
    out_type=(jax.ShapeDtypeStruct((NV * D // 2,), jnp.int32),
              jax.ShapeDtypeStruct((NV * D // 2,), jnp.int32)),
    scratch_types=[
        [pltpu.VMEM((D, BW), jnp.float32) for _ in range(2)],
        [pltpu.VMEM((BW * D // 2,), jnp.int32) for _ in range(2)],
        pltpu.VMEM((TAIL * D // 2,), jnp.int32),
        [pltpu.SemaphoreType.DMA for _ in range(2)],
        [pltpu.SemaphoreType.DMA for _ in range(2)],
    ],
)
def _sc_transpose(ut_hbm, vt_hbm, tu_hbm, tv_hbm, uout_hbm, vout_hbm,
                  slab, outb, tailb, sin, sout):
    wid = lax.axis_index("s") * NC + lax.axis_index("c")
    lane = lax.iota(jnp.int32, 16)

    def transpose16(vecs):
        for s in (1, 2, 4, 8):
            m = (lane & s) == 0
            nxt = list(vecs)
            for j in range(16):
                if j & s:
                    continue
                x, y = vecs[j], vecs[j | s]
                w = jnp.where(m, y, x)
                wg = w.at[lane ^ s].get(mode="promise_in_bounds")
                nxt[j] = jnp.where(m, x, wg)
                nxt[j | s] = jnp.where(m, wg, y)
            vecs = nxt
        return vecs

    def transpose_block(sl, ob):
        def group_body(g, c2):
            base = g * 16
            cols0 = transpose16([sl[d, pl.ds(base, 16)] for d in range(16)])
            cols1 = transpose16([sl[16 + d, pl.ds(base, 16)]
                                 for d in range(16)])
            for j in range(16):
                packed = plsc.pack(cols0[j], cols1[j],
                                   format=plsc.PackFormat.INTERLEAVED)
                ob[pl.ds((base + j) * (D // 2), D // 2)] = plsc.bitcast(
                    packed, jnp.int32)
            return c2

        lax.fori_loop(0, BW // 16, group_body, 0)

    def table_pass(tab_hbm, out_hbm):
        def in_copy(kk, par):
            b = wid + NW * kk
            return pltpu.make_async_copy(
                tab_hbm.at[:, pl.ds(b * BW, BW)], slab[par], sin[par])

        def out_copy(kk, par):
            b = wid + NW * kk
            off = pl.multiple_of(b * (BW * D // 2), 8)
            return pltpu.make_async_copy(
                outb[par], out_hbm.at[pl.ds(off, BW * D // 2)], sout[par])

        in_copy(0, 0).start()

        def it(kk, par):
            in_copy(kk, par).wait()

            @pl.when(kk + 1 < NB_W)
            def _():
                in_copy(kk + 1, 1 - par).start()

            @pl.when(kk >= 2)
            def _():
                out_copy(kk - 2, par).wait()

            transpose_block(slab[par], outb[par])
            out_copy(kk, par).start()

        def outer(i, carry):
            it(i * 2, 0)
            it(i * 2 + 1, 1)
            return carry

        lax.fori_loop(0, NB_W // 2, outer, 0)
        out_copy(NB_W - 2, 0).wait()
        out_copy(NB_W - 1, 1).wait()

    table_pass(ut_hbm, uout_hbm)
    table_pass(vt_hbm, vout_hbm)

    @pl.when(wid == NW - 1)
    def _():
        pltpu.sync_copy(tu_hbm, tailb)
        pltpu.sync_copy(
            tailb, uout_hbm.at[pl.ds(MAIN * D // 2, TAIL * D // 2)])
        pltpu.sync_copy(tv_hbm, tailb)
        pltpu.sync_copy(
            tailb, vout_hbm.at[pl.ds(MAIN * D // 2, TAIL * D // 2)])


@functools.partial(
    pl.kernel,
    mesh=_mesh,
    compiler_params=pltpu.CompilerParams(use_tc_tiling_on_sc=False,
                                         needs_layout_passes=False),
    out_type=jax.ShapeDtypeStruct((NNZ,), jnp.float32),
    scratch_types=[
        [pltpu.VMEM((C,), jnp.int32) for _ in range(4)],
        [pltpu.VMEM((C,), jnp.int32) for _ in range(4)],
        [pltpu.VMEM((C, D // 2), jnp.int32) for _ in range(2)],
        [pltpu.VMEM((C, D // 2), jnp.int32) for _ in range(2)],
        pltpu.VMEM((PER_W,), jnp.float32),
        [pltpu.SemaphoreType.DMA for _ in range(2)],
        [pltpu.SemaphoreType.DMA for _ in range(4)],
    ],
)
def _sc_dot_gather(indices_hbm, u_hbm, v_hbm, out_hbm,
                   idxu, idxv, ru, rv, outw, semg, semi):
    wid = lax.axis_index("s") * NC + lax.axis_index("c")
    base = wid * PER_W

    lane = lax.iota(jnp.int32, 16)
    bitrev = (((lane & 1) << 3) | ((lane & 2) << 1)
              | ((lane & 4) >> 1) | ((lane & 8) >> 3))

    def idx_copy(k, slot, sem):
        off = base + k * C
        a = pltpu.async_copy(indices_hbm.at[pl.ds(off, C)], idxu[slot], sem)
        b = pltpu.async_copy(indices_hbm.at[pl.ds(NNZ + off, C)],
                             idxv[slot], sem)
        return a, b

    for k in (0, 1):
        a, b = idx_copy(k, k, semi[k])
        a.wait()
        b.wait()
    idx_copy(2, 2, semi[2])
    for k in (0, 1):
        pltpu.async_copy(u_hbm.at[idxu[k]], ru[k], semg[k])
        pltpu.async_copy(v_hbm.at[idxv[k]], rv[k], semg[k])

    BR = [0, 8, 4, 12, 2, 10, 6, 14, 1, 9, 5, 13, 3, 11, 7, 15]

    def compute_chunk(rup, rvp, out_base):
        def row_prod(r):
            uw = rup[r, pl.ds(0, D // 2)]
            vw = rvp[r, pl.ds(0, D // 2)]
            u0 = plsc.bitcast(uw << 16, jnp.float32)
            u1 = plsc.bitcast(uw & jnp.int32(-65536), jnp.float32)
            v0 = plsc.bitcast(vw << 16, jnp.float32)
            v1 = plsc.bitcast(vw & jnp.int32(-65536), jnp.float32)
            return u0 * v0 + u1 * v1

        def group_body(g, c2):
            r0 = g * 16
            vecs = [row_prod(r0 + j) for j in BR]
            for h in (8, 4, 2, 1):
                m = (lane & h) == 0
                nm = (lane & h) != 0
                nxt = []
                for i in range(0, len(vecs), 2):
                    a, b = vecs[i], vecs[i + 1]
                    w = jnp.where(nm, a, b)
                    nxt.append(jnp.where(m, a, b)
                               + w.at[lane ^ h].get(mode="promise_in_bounds"))
                vecs = nxt
            outw[pl.ds(out_base + r0, 16)] = vecs[0]
            return c2

        lax.fori_loop(0, C // 16, group_body, 0)

    def iteration(k, b):
        p = b % 2
        r2 = (b + 2) % 4
        r3 = (b + 3) % 4
        pltpu.make_async_copy(u_hbm.at[idxu[b]], ru[p], semg[p]).wait()
        pltpu.make_async_copy(v_hbm.at[idxv[b]], rv[p], semg[p]).wait()

        compute_chunk(ru[p], rv[p], k * C)

        @pl.when(k + 2 < NCHUNK)
        def _():
            pltpu.make_async_copy(
                indices_hbm.at[pl.ds(base, C)], idxu[r2], semi[r2]).wait()
            pltpu.make_async_copy(
                indices_hbm.at[pl.ds(base, C)], idxv[r2], semi[r2]).wait()
            pltpu.async_copy(u_hbm.at[idxu[r2]], ru[p], semg[p])
            pltpu.async_copy(v_hbm.at[idxv[r2]], rv[p], semg[p])

        @pl.when(k + 3 < NCHUNK)
        def _():
            idx_copy(k + 3, r3, semi[r3])

    def outer(i, carry):
        for b in range(4):
            iteration(i * 4 + b, b)
        return carry

    lax.fori_loop(0, NCHUNK // 4, outer, 0)
    pltpu.sync_copy(outw, out_hbm.at[pl.ds(base, PER_W)])


_PERM = [c for p in zip(range(16), range(16, 32)) for c in p]


def kernel(indices, U, V):
    ut, vt = U.T, V.T
    def tail_words(T):
        tb = T[MAIN:][:, jnp.array(_PERM)].astype(jnp.bfloat16)
        return lax.bitcast_convert_type(
            tb.reshape(TAIL * D // 2, 2), jnp.int32)

    uflat, vflat = _sc_transpose(ut, vt, tail_words(U), tail_words(V))
    return _sc_dot_gather(indices.reshape(-1), uflat.reshape(NV, D // 2),
                          vflat.reshape(NV, D // 2))


# speedup vs baseline: 1.0798x; 1.0798x over previous
import functools

import jax
import jax.numpy as jnp
from jax import lax
from jax.experimental import pallas as pl
from jax.experimental.pallas import tpu as pltpu
from jax.experimental.pallas import tpu_sc as plsc

NNZ = 2097152
D = 32
NV = 1000000
NC = 2
NS = 16
NW = NC * NS
PER_W = NNZ // NW
C = 256
NCHUNK = PER_W // C

BW = 256
NBLK = 3904
MAIN = NBLK * BW
TAIL = NV - MAIN
NB_W = NBLK // NW

_mesh = plsc.VectorSubcoreMesh(core_axis_name="c", subcore_axis_name="s")


@functools.partial(
    pl.kernel,
    mesh=_mesh,
    compiler_params=pltpu.CompilerParams(needs_layout_passes=False),
    out_type=(jax.ShapeDtypeStruct((NV * D // 2,), jnp.int32),
              jax.ShapeDtypeStruct((NV * D // 2,), jnp.int32)),
    scratch_types=[
        [pltpu.VMEM((D, BW), jnp.float32) for _ in range(2)],
        [pltpu.VMEM((BW * D // 2,), jnp.int32) for _ in range(2)],
        pltpu.VMEM((TAIL * D // 2,), jnp.int32),
        [pltpu.SemaphoreType.DMA for _ in range(2)],
        [pltpu.SemaphoreType.DMA for _ in range(2)],
    ],
)
def _sc_transpose(ut_hbm, vt_hbm, tu_hbm, tv_hbm, uout_hbm, vout_hbm,
                  slab, outb, tailb, sin, sout):
    wid = lax.axis_index("s") * NC + lax.axis_index("c")
    lane = lax.iota(jnp.int32, 16)

    def transpose16(vecs):
        for s in (1, 2, 4, 8):
            m = (lane & s) == 0
            nxt = list(vecs)
            for j in range(16):
                if j & s:
                    continue
                x, y = vecs[j], vecs[j | s]
                w = jnp.where(m, y, x)
                wg = w.at[lane ^ s].get(mode="promise_in_bounds")
                nxt[j] = jnp.where(m, x, wg)
                nxt[j | s] = jnp.where(m, wg, y)
            vecs = nxt
        return vecs

    def transpose_block(sl, ob):
        def group_body(g, c2):
            base = g * 16
            cols0 = transpose16([sl[d, pl.ds(base, 16)] for d in range(16)])
            cols1 = transpose16([sl[16 + d, pl.ds(base, 16)]
                                 for d in range(16)])
            for j in range(16):
                packed = plsc.pack(cols0[j], cols1[j],
                                   format=plsc.PackFormat.INTERLEAVED)
                ob[pl.ds((base + j) * (D // 2), D // 2)] = plsc.bitcast(
                    packed, jnp.int32)
            return c2

        lax.fori_loop(0, BW // 16, group_body, 0)

    def table_pass(tab_hbm, out_hbm):
        def in_copy(kk, par):
            b = wid + NW * kk
            return pltpu.make_async_copy(
                tab_hbm.at[:, pl.ds(b * BW, BW)], slab[par], sin[par])

        def out_copy(kk, par):
            b = wid + NW * kk
            off = pl.multiple_of(b * (BW * D // 2), 8)
            return pltpu.make_async_copy(
                outb[par], out_hbm.at[pl.ds(off, BW * D // 2)], sout[par])

        in_copy(0, 0).start()

        def it(kk, par):
            in_copy(kk, par).wait()

            @pl.when(kk + 1 < NB_W)
            def _():
                in_copy(kk + 1, 1 - par).start()

            @pl.when(kk >= 2)
            def _():
                out_copy(kk - 2, par).wait()

            transpose_block(slab[par], outb[par])
            out_copy(kk, par).start()

        def outer(i, carry):
            it(i * 2, 0)
            it(i * 2 + 1, 1)
            return carry

        lax.fori_loop(0, NB_W // 2, outer, 0)
        out_copy(NB_W - 2, 0).wait()
        out_copy(NB_W - 1, 1).wait()

    table_pass(ut_hbm, uout_hbm)
    table_pass(vt_hbm, vout_hbm)

    @pl.when(wid == NW - 1)
    def _():
        pltpu.sync_copy(tu_hbm, tailb)
        pltpu.sync_copy(
            tailb, uout_hbm.at[pl.ds(MAIN * D // 2, TAIL * D // 2)])
        pltpu.sync_copy(tv_hbm, tailb)
        pltpu.sync_copy(
            tailb, vout_hbm.at[pl.ds(MAIN * D // 2, TAIL * D // 2)])


@functools.partial(
    pl.kernel,
    mesh=_mesh,
    compiler_params=pltpu.CompilerParams(use_tc_tiling_on_sc=False,
                                         needs_layout_passes=False),
    out_type=jax.ShapeDtypeStruct((NNZ,), jnp.float32),
    scratch_types=[
        [pltpu.VMEM((C,), jnp.int32) for _ in range(8)],
        [pltpu.VMEM((C,), jnp.int32) for _ in range(8)],
        [pltpu.VMEM((C, D // 2), jnp.int32) for _ in range(4)],
        [pltpu.VMEM((C, D // 2), jnp.int32) for _ in range(4)],
        pltpu.VMEM((PER_W,), jnp.float32),
        [pltpu.SemaphoreType.DMA for _ in range(4)],
        [pltpu.SemaphoreType.DMA for _ in range(8)],
    ],
)
def _sc_dot_gather(indices_hbm, u_hbm, v_hbm, out_hbm,
                   idxu, idxv, ru, rv, outw, semg, semi):
    wid = lax.axis_index("s") * NC + lax.axis_index("c")
    base = wid * PER_W

    lane = lax.iota(jnp.int32, 16)
    bitrev = (((lane & 1) << 3) | ((lane & 2) << 1)
              | ((lane & 4) >> 1) | ((lane & 8) >> 3))

    def idx_copy(k, slot, sem):
        off = base + k * C
        a = pltpu.async_copy(indices_hbm.at[pl.ds(off, C)], idxu[slot], sem)
        b = pltpu.async_copy(indices_hbm.at[pl.ds(NNZ + off, C)],
                             idxv[slot], sem)
        return a, b

    for k in range(4):
        a, b = idx_copy(k, k, semi[k])
        a.wait()
        b.wait()
    idx_copy(4, 4, semi[4])
    for k in range(4):
        pltpu.async_copy(u_hbm.at[idxu[k]], ru[k], semg[k])
        pltpu.async_copy(v_hbm.at[idxv[k]], rv[k], semg[k])

    BR = [0, 8, 4, 12, 2, 10, 6, 14, 1, 9, 5, 13, 3, 11, 7, 15]

    def compute_chunk(rup, rvp, out_base):
        def row_prod(r):
            uw = rup[r, pl.ds(0, D // 2)]
            vw = rvp[r, pl.ds(0, D // 2)]
            u0 = plsc.bitcast(uw << 16, jnp.float32)
            u1 = plsc.bitcast(uw & jnp.int32(-65536), jnp.float32)
            v0 = plsc.bitcast(vw << 16, jnp.float32)
            v1 = plsc.bitcast(vw & jnp.int32(-65536), jnp.float32)
            return u0 * v0 + u1 * v1

        def group_body(g, c2):
            r0 = g * 16
            vecs = [row_prod(r0 + j) for j in BR]
            for h in (8, 4, 2, 1):
                m = (lane & h) == 0
                nm = (lane & h) != 0
                nxt = []
                for i in range(0, len(vecs), 2):
                    a, b = vecs[i], vecs[i + 1]
                    w = jnp.where(nm, a, b)
                    nxt.append(jnp.where(m, a, b)
                               + w.at[lane ^ h].get(mode="promise_in_bounds"))
                vecs = nxt
            outw[pl.ds(out_base + r0, 16)] = vecs[0]
            return c2

        lax.fori_loop(0, C // 16, group_body, 0)

    def iteration(k, b8):
        p = b8 % 4
        r4 = (b8 + 4) % 8
        r5 = (b8 + 5) % 8
        pltpu.make_async_copy(u_hbm.at[idxu[b8]], ru[p], semg[p]).wait()
        pltpu.make_async_copy(v_hbm.at[idxv[b8]], rv[p], semg[p]).wait()

        compute_chunk(ru[p], rv[p], k * C)

        @pl.when(k + 4 < NCHUNK)
        def _():
            pltpu.make_async_copy(
                indices_hbm.at[pl.ds(base, C)], idxu[r4], semi[r4]).wait()
            pltpu.make_async_copy(
                indices_hbm.at[pl.ds(base, C)], idxv[r4], semi[r4]).wait()
            pltpu.async_copy(u_hbm.at[idxu[r4]], ru[p], semg[p])
            pltpu.async_copy(v_hbm.at[idxv[r4]], rv[p], semg[p])

        @pl.when(k + 5 < NCHUNK)
        def _():
            idx_copy(k + 5, r5, semi[r5])

    def outer(i, carry):
        for b8 in range(8):
            iteration(i * 8 + b8, b8)
        return carry

    lax.fori_loop(0, NCHUNK // 8, outer, 0)
    pltpu.sync_copy(outw, out_hbm.at[pl.ds(base, PER_W)])


_PERM = [c for p in zip(range(16), range(16, 32)) for c in p]


def kernel(indices, U, V):
    ut, vt = U.T, V.T
    def tail_words(T):
        tb = T[MAIN:][:, jnp.array(_PERM)].astype(jnp.bfloat16)
        return lax.bitcast_convert_type(
            tb.reshape(TAIL * D // 2, 2), jnp.int32)

    uflat, vflat = _sc_transpose(ut, vt, tail_words(U), tail_words(V))
    return _sc_dot_gather(indices.reshape(-1), uflat.reshape(NV, D // 2),
                          vflat.reshape(NV, D // 2))


# speedup vs baseline: 1.1241x; 1.0410x over previous
import functools

import jax
import jax.numpy as jnp
from jax import lax
from jax.experimental import pallas as pl
from jax.experimental.pallas import tpu as pltpu
from jax.experimental.pallas import tpu_sc as plsc

NNZ = 2097152
D = 32
NV = 1000000
NC = 2
NS = 16
NW = NC * NS
PER_W = NNZ // NW
C = 256
NCHUNK = PER_W // C

BW = 256
NBLK = 3904
MAIN = NBLK * BW
TAIL = NV - MAIN
NB_W = NBLK // NW

_mesh = plsc.VectorSubcoreMesh(core_axis_name="c", subcore_axis_name="s")


@functools.partial(
    pl.kernel,
    mesh=_mesh,
    compiler_params=pltpu.CompilerParams(needs_layout_passes=False),
    out_type=(jax.ShapeDtypeStruct((NV * D // 2,), jnp.int32),
              jax.ShapeDtypeStruct((NV * D // 2,), jnp.int32)),
    scratch_types=[
        [pltpu.VMEM((D, BW), jnp.float32) for _ in range(4)],
        [pltpu.VMEM((BW * D // 2,), jnp.int32) for _ in range(2)],
        pltpu.VMEM((TAIL * D // 2,), jnp.int32),
        [pltpu.SemaphoreType.DMA for _ in range(4)],
        [pltpu.SemaphoreType.DMA for _ in range(2)],
    ],
)
def _sc_transpose(ut_hbm, vt_hbm, tu_hbm, tv_hbm, uout_hbm, vout_hbm,
                  slab, outb, tailb, sin, sout):
    wid = lax.axis_index("s") * NC + lax.axis_index("c")
    lane = lax.iota(jnp.int32, 16)

    def transpose16(vecs):
        for s in (1, 2, 4, 8):
            m = (lane & s) == 0
            nxt = list(vecs)
            for j in range(16):
                if j & s:
                    continue
                x, y = vecs[j], vecs[j | s]
                w = jnp.where(m, y, x)
                wg = w.at[lane ^ s].get(mode="promise_in_bounds")
                nxt[j] = jnp.where(m, x, wg)
                nxt[j | s] = jnp.where(m, wg, y)
            vecs = nxt
        return vecs

    def transpose_block(sl, ob):
        def group_body(g, c2):
            base = g * 16
            cols0 = transpose16([sl[d, pl.ds(base, 16)] for d in range(16)])
            cols1 = transpose16([sl[16 + d, pl.ds(base, 16)]
                                 for d in range(16)])
            for j in range(16):
                packed = plsc.pack(cols0[j], cols1[j],
                                   format=plsc.PackFormat.INTERLEAVED)
                ob[pl.ds((base + j) * (D // 2), D // 2)] = plsc.bitcast(
                    packed, jnp.int32)
            return c2

        lax.fori_loop(0, BW // 16, group_body, 0)

    def table_pass(tab_hbm, out_hbm):
        def in_copy(kk, par):
            b = wid + NW * kk
            return pltpu.make_async_copy(
                tab_hbm.at[:, pl.ds(b * BW, BW)], slab[par], sin[par])

        def out_copy(kk, par):
            b = wid + NW * kk
            off = pl.multiple_of(b * (BW * D // 2), 8)
            return pltpu.make_async_copy(
                outb[par], out_hbm.at[pl.ds(off, BW * D // 2)], sout[par])

        for kk in range(3):
            in_copy(kk, kk).start()

        def it(kk, s4, par):
            in_copy(kk, s4).wait()

            @pl.when(kk + 3 < NB_W)
            def _():
                in_copy(kk + 3, (s4 + 3) % 4).start()

            @pl.when(kk >= 2)
            def _():
                out_copy(kk - 2, par).wait()

            transpose_block(slab[s4], outb[par])
            out_copy(kk, par).start()

        def outer(i, carry):
            for b in range(4):
                it(i * 4 + b, b, b % 2)
            return carry

        lax.fori_loop(0, NB_W // 4, outer, 0)
        it(NB_W - 2, (NB_W - 2) % 4, 0)
        it(NB_W - 1, (NB_W - 1) % 4, 1)
        out_copy(NB_W - 2, 0).wait()
        out_copy(NB_W - 1, 1).wait()

    table_pass(ut_hbm, uout_hbm)
    table_pass(vt_hbm, vout_hbm)

    @pl.when(wid == NW - 1)
    def _():
        pltpu.sync_copy(tu_hbm, tailb)
        pltpu.sync_copy(
            tailb, uout_hbm.at[pl.ds(MAIN * D // 2, TAIL * D // 2)])
        pltpu.sync_copy(tv_hbm, tailb)
        pltpu.sync_copy(
            tailb, vout_hbm.at[pl.ds(MAIN * D // 2, TAIL * D // 2)])


@functools.partial(
    pl.kernel,
    mesh=_mesh,
    compiler_params=pltpu.CompilerParams(use_tc_tiling_on_sc=False,
                                         needs_layout_passes=False),
    out_type=jax.ShapeDtypeStruct((NNZ,), jnp.float32),
    scratch_types=[
        [pltpu.VMEM((C,), jnp.int32) for _ in range(8)],
        [pltpu.VMEM((C,), jnp.int32) for _ in range(8)],
        [pltpu.VMEM((C, D // 2), jnp.int32) for _ in range(4)],
        [pltpu.VMEM((C, D // 2), jnp.int32) for _ in range(4)],
        pltpu.VMEM((PER_W,), jnp.float32),
        [pltpu.SemaphoreType.DMA for _ in range(4)],
        [pltpu.SemaphoreType.DMA for _ in range(8)],
    ],
)
def _sc_dot_gather(indices_hbm, u_hbm, v_hbm, out_hbm,
                   idxu, idxv, ru, rv, outw, semg, semi):
    wid = lax.axis_index("s") * NC + lax.axis_index("c")
    base = wid * PER_W

    lane = lax.iota(jnp.int32, 16)
    bitrev = (((lane & 1) << 3) | ((lane & 2) << 1)
              | ((lane & 4) >> 1) | ((lane & 8) >> 3))

    def idx_copy(k, slot, sem):
        off = base + k * C
        a = pltpu.async_copy(indices_hbm.at[pl.ds(off, C)], idxu[slot], sem)
        b = pltpu.async_copy(indices_hbm.at[pl.ds(NNZ + off, C)],
                             idxv[slot], sem)
        return a, b

    for k in range(4):
        a, b = idx_copy(k, k, semi[k])
        a.wait()
        b.wait()
    idx_copy(4, 4, semi[4])
    for k in range(4):
        pltpu.async_copy(u_hbm.at[idxu[k]], ru[k], semg[k])
        pltpu.async_copy(v_hbm.at[idxv[k]], rv[k], semg[k])

    BR = [0, 8, 4, 12, 2, 10, 6, 14, 1, 9, 5, 13, 3, 11, 7, 15]

    def compute_chunk(rup, rvp, out_base):
        def row_prod(r):
            uw = rup[r, pl.ds(0, D // 2)]
            vw = rvp[r, pl.ds(0, D // 2)]
            u0 = plsc.bitcast(uw << 16, jnp.float32)
            u1 = plsc.bitcast(uw & jnp.int32(-65536), jnp.float32)
            v0 = plsc.bitcast(vw << 16, jnp.float32)
            v1 = plsc.bitcast(vw & jnp.int32(-65536), jnp.float32)
            return u0 * v0 + u1 * v1

        def group_body(g, c2):
            r0 = g * 16
            vecs = [row_prod(r0 + j) for j in BR]
            for h in (8, 4, 2, 1):
                m = (lane & h) == 0
                nm = (lane & h) != 0
                nxt = []
                for i in range(0, len(vecs), 2):
                    a, b = vecs[i], vecs[i + 1]
                    w = jnp.where(nm, a, b)
                    nxt.append(jnp.where(m, a, b)
                               + w.at[lane ^ h].get(mode="promise_in_bounds"))
                vecs = nxt
            outw[pl.ds(out_base + r0, 16)] = vecs[0]
            return c2

        lax.fori_loop(0, C // 16, group_body, 0)

    def iteration(k, b8):
        p = b8 % 4
        r4 = (b8 + 4) % 8
        r5 = (b8 + 5) % 8
        pltpu.make_async_copy(u_hbm.at[idxu[b8]], ru[p], semg[p]).wait()
        pltpu.make_async_copy(v_hbm.at[idxv[b8]], rv[p], semg[p]).wait()

        compute_chunk(ru[p], rv[p], k * C)

        @pl.when(k + 4 < NCHUNK)
        def _():
            pltpu.make_async_copy(
                indices_hbm.at[pl.ds(base, C)], idxu[r4], semi[r4]).wait()
            pltpu.make_async_copy(
                indices_hbm.at[pl.ds(base, C)], idxv[r4], semi[r4]).wait()
            pltpu.async_copy(u_hbm.at[idxu[r4]], ru[p], semg[p])
            pltpu.async_copy(v_hbm.at[idxv[r4]], rv[p], semg[p])

        @pl.when(k + 5 < NCHUNK)
        def _():
            idx_copy(k + 5, r5, semi[r5])

    def outer(i, carry):
        for b8 in range(8):
            iteration(i * 8 + b8, b8)
        return carry

    lax.fori_loop(0, NCHUNK // 8, outer, 0)
    pltpu.sync_copy(outw, out_hbm.at[pl.ds(base, PER_W)])


_PERM = [c for p in zip(range(16), range(16, 32)) for c in p]


def kernel(indices, U, V):
    ut, vt = U.T, V.T
    def tail_words(T):
        tb = T[MAIN:][:, jnp.array(_PERM)].astype(jnp.bfloat16)
        return lax.bitcast_convert_type(
            tb.reshape(TAIL * D // 2, 2), jnp.int32)

    uflat, vflat = _sc_transpose(ut, vt, tail_words(U), tail_words(V))
    return _sc_dot_gather(indices.reshape(-1), uflat.reshape(NV, D // 2),
                          vflat.reshape(NV, D // 2))
